# baseline (speedup 1.0000x reference)
import functools

import jax
import jax.numpy as jnp
from jax import lax
from jax.experimental import pallas as pl
from jax.experimental.pallas import tpu as pltpu
from jax.experimental.pallas import tpu_sc as plsc

N = 10000
D = 128
E = 320000

NC = 2
NS = 16
L = 16
NP = 10240
SB = 160
BB = SB // NC
HB = 2 * BB
W = 4
NWB = BB // W
EP = NS * SB * 128
RPT = NP // NS
NB = EP // 128
NB64 = EP // 64


def _newton_rsqrt(s):
    s = jnp.maximum(s, 1e-24)
    i = lax.bitcast_convert_type(s, jnp.int32)
    i = jnp.int32(0x5F3759DF) - lax.shift_right_logical(i, 1)
    y = lax.bitcast_convert_type(i, jnp.float32)
    for _ in range(4):
        y = y * (1.5 - 0.5 * s * y * y)
    return y


def _xn_body(x_ref, o_ref):
    xb = x_ref[...]
    s = jnp.sum(xb * xb, axis=1, keepdims=True)
    o_ref[...] = xb * lax.rsqrt(jnp.maximum(s, 1e-24))


def _combine_body(ws_ref, wi_ref, xn_ref, p0_ref, p1_ref, o_ref):
    o_ref[...] = (ws_ref[...] * xn_ref[...]
                  + wi_ref[...] * (p0_ref[...] + p1_ref[...]))


def _sc_body(row_hbm, col_hbm, ea_hbm, beta_hbm, eps_hbm, xn_hbm,
             p0_hbm, p1_hbm, ws_hbm, wi_hbm,
             row_win, ea_win, col_win, rsq_v, slice_v, wbuf,
             beta_v, eps_v, gbuf,
             sq_s, asum_s, acc_s, gsem, wsem, ssem):
    c = lax.axis_index("c")
    s = lax.axis_index("s")
    base = s * RPT
    abase = s * SB
    gbase = s * SB + c * BB

    pltpu.sync_copy(beta_hbm, beta_v)
    pltpu.sync_copy(eps_hbm, eps_v)
    bvec = beta_v[...]
    ebeta = jnp.exp(bvec)

    for k in range(RPT // L):
        slice_v[pl.ds(k * L, L)] = jnp.zeros((L,), jnp.float32)
    pltpu.sync_copy(slice_v, sq_s.at[pl.ds(base, RPT)])

    def _zrow(j, carry):
        for k in range(D // L):
            gbuf[0, j, pl.ds(k * L, L)] = jnp.zeros((L,), jnp.float32)
        return carry
    lax.fori_loop(0, 64, _zrow, 0)
    for t in range(RPT // 64):
        pltpu.sync_copy(gbuf.at[0], acc_s.at[pl.ds(base + t * 64, 64)])

    for k in range(RPT // L):
        slice_v[pl.ds(k * L, L)] = ebeta
    pltpu.sync_copy(slice_v, asum_s.at[pl.ds(base, RPT)])
    plsc.subcore_barrier()

    pltpu.sync_copy(row_hbm.at[pl.ds(2 * abase, 2 * W)], row_win.at[0])
    pltpu.sync_copy(ea_hbm.at[pl.ds(abase, W)], ea_win.at[0])

    def _a1(t, carry):
        p = t % 2

        @pl.when(t + 1 < SB // W)
        def _():
            pltpu.async_copy(row_hbm.at[pl.ds(2 * (abase + (t + 1) * W),
                                              2 * W)],
                             row_win.at[1 - p], wsem)
            pltpu.async_copy(ea_hbm.at[pl.ds(abase + (t + 1) * W, W)],
                             ea_win.at[1 - p], wsem)
        for b in range(W):
            for g in range(8):
                eav = ea_win[p, b, pl.ds(g * L, L)]
                wbuf[b, pl.ds(g * L, L)] = eav * eav
        for b in range(W):
            for h in range(2):
                pltpu.async_copy(wbuf.at[b, pl.ds(h * 64, 64)],
                                 sq_s.at[row_win.at[p, 2 * b + h]], ssem,
                                 add=True)
        for b in range(W):
            for h in range(2):
                pltpu.make_async_copy(wbuf.at[b, pl.ds(h * 64, 64)],
                                      sq_s.at[row_win.at[p, 2 * b + h]],
                                      ssem).wait()

        @pl.when(t + 1 < SB // W)
        def _():
            pltpu.make_async_copy(row_hbm.at[pl.ds(2 * (abase + (t + 1) * W),
                                                   2 * W)],
                                  row_win.at[1 - p], wsem).wait()
            pltpu.make_async_copy(ea_hbm.at[pl.ds(abase + (t + 1) * W, W)],
                                  ea_win.at[1 - p], wsem).wait()
        return carry
    lax.fori_loop(0, SB // W, _a1, 0)
    plsc.subcore_barrier()

    pltpu.sync_copy(sq_s.at[pl.ds(base, RPT)], slice_v)
    for k in range(RPT // L):
        slice_v[pl.ds(k * L, L)] = _newton_rsqrt(slice_v[pl.ds(k * L, L)])
    pltpu.sync_copy(slice_v, sq_s.at[pl.ds(base, RPT)])
    plsc.subcore_barrier()
    pltpu.sync_copy(sq_s, rsq_v)

    pltpu.sync_copy(row_hbm.at[pl.ds(2 * abase, 2 * W)], row_win.at[0])
    pltpu.sync_copy(ea_hbm.at[pl.ds(abase, W)], ea_win.at[0])

    def _a2(t, carry):
        p = t % 2

        @pl.when(t + 1 < SB // W)
        def _():
            pltpu.async_copy(row_hbm.at[pl.ds(2 * (abase + (t + 1) * W),
                                              2 * W)],
                             row_win.at[1 - p], wsem)
            pltpu.async_copy(ea_hbm.at[pl.ds(abase + (t + 1) * W, W)],
                             ea_win.at[1 - p], wsem)
        for b in range(W):
            for g in range(8):
                rowv = row_win[p, 2 * b + g // 4, pl.ds((g % 4) * L, L)]
                eav = ea_win[p, b, pl.ds(g * L, L)]
                rsqv = plsc.load_gather(rsq_v, [rowv])
                wbuf[b, pl.ds(g * L, L)] = jnp.exp(bvec * eav * rsqv)
        for b in range(W):
            for h in range(2):
                pltpu.async_copy(wbuf.at[b, pl.ds(h * 64, 64)],
                                 asum_s.at[row_win.at[p, 2 * b + h]], ssem,
                                 add=True)
        for b in range(W):
            for h in range(2):
                pltpu.make_async_copy(wbuf.at[b, pl.ds(h * 64, 64)],
                                      asum_s.at[row_win.at[p, 2 * b + h]],
                                      ssem).wait()

        @pl.when(t + 1 < SB // W)
        def _():
            pltpu.make_async_copy(row_hbm.at[pl.ds(2 * (abase + (t + 1) * W),
                                                   2 * W)],
                                  row_win.at[1 - p], wsem).wait()
            pltpu.make_async_copy(ea_hbm.at[pl.ds(abase + (t + 1) * W, W)],
                                  ea_win.at[1 - p], wsem).wait()
        return carry
    lax.fori_loop(0, SB // W, _a2, 0)
    plsc.subcore_barrier()

    pltpu.sync_copy(asum_s.at[pl.ds(base, RPT)], slice_v)
    epsv = eps_v[...]
    for k in range(RPT // L):
        slice_v[pl.ds(k * L, L)] = 1.0 / (slice_v[pl.ds(k * L, L)] + 1e-16)

    @pl.when(c == 0)
    def _():
        pltpu.sync_copy(slice_v, wi_hbm.at[pl.ds(base, RPT)])
    for k in range(RPT // L):
        slice_v[pl.ds(k * L, L)] = (1.0 + epsv) + ebeta * slice_v[pl.ds(k * L, L)]

    @pl.when(c == 0)
    def _():
        pltpu.sync_copy(slice_v, ws_hbm.at[pl.ds(base, RPT)])

    pltpu.sync_copy(row_hbm.at[pl.ds(2 * gbase, 2 * W)], row_win.at[0])
    pltpu.sync_copy(ea_hbm.at[pl.ds(gbase, W)], ea_win.at[0])
    pltpu.sync_copy(col_hbm.at[pl.ds(gbase, W)], col_win.at[0])
    pltpu.async_copy(xn_hbm.at[col_win.at[0, 0, pl.ds(0, 64)]],
                     gbuf.at[0], gsem)

    def _bsub(hb, carry):
        t = hb // 8
        pos = hb % 8
        pt = t % 3
        r = hb % 3
        b = pos // 2
        h = pos % 2

        @pl.when((pos == 0) & (t + 1 < NWB))
        def _():
            nxt = pl.ds(gbase + (t + 1) * W, W)
            pltpu.async_copy(row_hbm.at[pl.ds(2 * (gbase + (t + 1) * W),
                                              2 * W)],
                             row_win.at[(t + 1) % 3], wsem)
            pltpu.async_copy(ea_hbm.at[nxt], ea_win.at[(t + 1) % 3], wsem)
            pltpu.async_copy(col_hbm.at[nxt], col_win.at[(t + 1) % 3], wsem)

        @pl.when((pos == 7) & (t + 1 < NWB))
        def _():
            nxt = pl.ds(gbase + (t + 1) * W, W)
            pltpu.make_async_copy(row_hbm.at[pl.ds(2 * (gbase + (t + 1) * W),
                                                   2 * W)],
                                  row_win.at[(t + 1) % 3], wsem).wait()
            pltpu.make_async_copy(ea_hbm.at[nxt], ea_win.at[(t + 1) % 3],
                                  wsem).wait()
            pltpu.make_async_copy(col_hbm.at[nxt], col_win.at[(t + 1) % 3],
                                  wsem).wait()

        @pl.when(hb + 1 < HB)
        def _():
            hn = hb + 1
            tn = hn // 8
            posn = hn % 8
            colref = col_win.at[tn % 3, posn // 2, pl.ds((posn % 2) * 64, 64)]
            pltpu.async_copy(xn_hbm.at[colref], gbuf.at[hn % 3], gsem)

        for g in range(4):
            rowv = row_win[pt, pos, pl.ds(g * L, L)]
            eav = ea_win[pt, b, pl.ds(h * 64 + g * L, L)]
            rsqv = plsc.load_gather(rsq_v, [rowv])
            wbuf[0, pl.ds(g * L, L)] = jnp.exp(bvec * eav * rsqv)

        pltpu.make_async_copy(xn_hbm.at[col_win.at[pt, b, pl.ds(h * 64, 64)]],
                              gbuf.at[r], gsem).wait()

        def _grpfn(g, carry2):
            wv = wbuf[0, pl.ds(g * L, L)]
            for u in range(L):
                wb = jnp.broadcast_to(wv[u], (L,))
                j = g * L + u
                for k in range(D // L):
                    gbuf[r, j, pl.ds(k * L, L)] = \
                        gbuf[r, j, pl.ds(k * L, L)] * wb
            return carry2
        lax.fori_loop(0, 4, _grpfn, 0)
        pltpu.sync_copy(gbuf.at[r], acc_s.at[row_win.at[pt, pos]],
                        add=True)
        return carry
    lax.fori_loop(0, HB, _bsub, 0)
    plsc.subcore_barrier()

    @pl.when(c == 0)
    def _():
        pltpu.sync_copy(acc_s.at[pl.ds(base, RPT)],
                        p0_hbm.at[pl.ds(base, RPT)])

    @pl.when(c == 1)
    def _():
        pltpu.sync_copy(acc_s.at[pl.ds(base, RPT)],
                        p1_hbm.at[pl.ds(base, RPT)])


_sc_kernel = functools.partial(
    pl.kernel,
    out_type=(
        jax.ShapeDtypeStruct((NP, D), jnp.float32),
        jax.ShapeDtypeStruct((NP, D), jnp.float32),
        jax.ShapeDtypeStruct((NP,), jnp.float32),
        jax.ShapeDtypeStruct((NP,), jnp.float32),
    ),
    mesh=plsc.VectorSubcoreMesh(core_axis_name="c", subcore_axis_name="s"),
    compiler_params=pltpu.CompilerParams(needs_layout_passes=False),
    scratch_types=[
        pltpu.VMEM((3, 2 * W, 64), jnp.int32),
        pltpu.VMEM((3, W, 128), jnp.float32),
        pltpu.VMEM((3, W, 128), jnp.int32),
        pltpu.VMEM((NP,), jnp.float32),
        pltpu.VMEM((RPT,), jnp.float32),
        pltpu.VMEM((W, 128), jnp.float32),
        pltpu.VMEM((L,), jnp.float32),
        pltpu.VMEM((L,), jnp.float32),
        pltpu.VMEM((3, 64, D), jnp.float32),
        pltpu.VMEM_SHARED((NP,), jnp.float32),
        pltpu.VMEM_SHARED((NP,), jnp.float32),
        pltpu.VMEM_SHARED((NP, D), jnp.float32),
        pltpu.SemaphoreType.DMA,
        pltpu.SemaphoreType.DMA,
        pltpu.SemaphoreType.DMA,
    ],
)(_sc_body)


@jax.jit
def kernel(x, edge_index, edge_attr, beta, eps):
    row = edge_index[0]
    col = edge_index[1]
    pad = EP - E
    pad_row = N + (jnp.arange(pad, dtype=jnp.int32) % (NP - N))
    pad_col = jnp.arange(pad, dtype=jnp.int32) % N
    row_p = jnp.concatenate([row, pad_row]).reshape(NB64, 64)
    col_p = jnp.concatenate([col, pad_col]).reshape(NB, 128)
    ea_p = jnp.concatenate(
        [edge_attr, jnp.zeros((pad,), jnp.float32)]).reshape(NB, 128)
    beta16 = jnp.broadcast_to(beta.astype(jnp.float32), (L,))
    eps16 = jnp.broadcast_to(eps.astype(jnp.float32), (L,))

    xn = pl.pallas_call(
        _xn_body,
        grid=(N // 1000,),
        in_specs=[pl.BlockSpec((1000, D), lambda i: (i, 0))],
        out_specs=pl.BlockSpec((1000, D), lambda i: (i, 0)),
        out_shape=jax.ShapeDtypeStruct((N, D), jnp.float32),
    )(x)

    p0, p1, wscale, winv = _sc_kernel(row_p, col_p, ea_p, beta16, eps16, xn)

    out = pl.pallas_call(
        _combine_body,
        grid=(N // 1000,),
        in_specs=[
            pl.BlockSpec((1000, 1), lambda i: (i, 0)),
            pl.BlockSpec((1000, 1), lambda i: (i, 0)),
            pl.BlockSpec((1000, D), lambda i: (i, 0)),
            pl.BlockSpec((1000, D), lambda i: (i, 0)),
            pl.BlockSpec((1000, D), lambda i: (i, 0)),
        ],
        out_specs=pl.BlockSpec((1000, D), lambda i: (i, 0)),
        out_shape=jax.ShapeDtypeStruct((N, D), jnp.float32),
    )(wscale.reshape(NP, 1), winv.reshape(NP, 1), xn, p0, p1)

    return out


# speedup vs baseline: 2.2881x; 2.2881x over previous
import functools

import jax
import jax.numpy as jnp
from jax import lax
from jax.experimental import pallas as pl
from jax.experimental.pallas import tpu as pltpu
from jax.experimental.pallas import tpu_sc as plsc

N = 10000
D = 128
E = 320000

NC = 2
NS = 16
L = 16
NP = 10240
SB = 160
BB = SB // NC
W = 4
NWB = BB // W
EP = NS * SB * 128
RPT = NP // NS
NB = EP // 128


def _newton_rsqrt(s):
    s = jnp.maximum(s, 1e-24)
    i = lax.bitcast_convert_type(s, jnp.int32)
    i = jnp.int32(0x5F3759DF) - lax.shift_right_logical(i, 1)
    y = lax.bitcast_convert_type(i, jnp.float32)
    for _ in range(4):
        y = y * (1.5 - 0.5 * s * y * y)
    return y


def _xn_body(x_ref, o_ref):
    xb = x_ref[...]
    s = jnp.sum(xb * xb, axis=1, keepdims=True)
    o_ref[...] = xb * lax.rsqrt(jnp.maximum(s, 1e-24))


def _combine_body(ws_ref, wi_ref, xn_ref, p0_ref, p1_ref, o_ref):
    o_ref[...] = (ws_ref[...] * xn_ref[...]
                  + wi_ref[...] * (p0_ref[...] + p1_ref[...]))


def _sc_body(row_hbm, col_hbm, ea_hbm, beta_hbm, eps_hbm, xn_hbm,
             p0_hbm, p1_hbm, ws_hbm, wi_hbm,
             row_win, ea_win, col_win, rsq_v, slice_v, wbuf,
             beta_v, eps_v, gbuf,
             sq_s, asum_s, acc_s, gsem, wsem, ssem):
    c = lax.axis_index("c")
    s = lax.axis_index("s")
    base = s * RPT
    abase = s * SB
    gbase = s * SB + c * BB

    pltpu.sync_copy(beta_hbm, beta_v)
    pltpu.sync_copy(eps_hbm, eps_v)
    bvec = beta_v[...]
    ebeta = jnp.exp(bvec)

    for k in range(RPT // L):
        slice_v[pl.ds(k * L, L)] = jnp.zeros((L,), jnp.float32)
    pltpu.sync_copy(slice_v, sq_s.at[pl.ds(base, RPT)])

    def _zrow(j, carry):
        for k in range(D // L):
            gbuf[0, j, pl.ds(k * L, L)] = jnp.zeros((L,), jnp.float32)
        return carry
    lax.fori_loop(0, 128, _zrow, 0)
    for t in range(RPT // 128):
        pltpu.sync_copy(gbuf.at[0], acc_s.at[pl.ds(base + t * 128, 128)])

    for k in range(RPT // L):
        slice_v[pl.ds(k * L, L)] = ebeta
    pltpu.sync_copy(slice_v, asum_s.at[pl.ds(base, RPT)])
    plsc.subcore_barrier()

    pltpu.sync_copy(row_hbm.at[pl.ds(abase, W)], row_win.at[0])
    pltpu.sync_copy(ea_hbm.at[pl.ds(abase, W)], ea_win.at[0])

    def _a1(t, carry):
        p = t % 2

        @pl.when(t + 1 < SB // W)
        def _():
            nxt = pl.ds(abase + (t + 1) * W, W)
            pltpu.async_copy(row_hbm.at[nxt], row_win.at[1 - p], wsem)
            pltpu.async_copy(ea_hbm.at[nxt], ea_win.at[1 - p], wsem)
        for b in range(W):
            for g in range(8):
                eav = ea_win[p, b, pl.ds(g * L, L)]
                wbuf[b, pl.ds(g * L, L)] = eav * eav
        for b in range(W):
            pltpu.async_copy(wbuf.at[b], sq_s.at[row_win.at[p, b]], ssem,
                             add=True)
        for b in range(W):
            pltpu.make_async_copy(wbuf.at[b], sq_s.at[row_win.at[p, b]],
                                  ssem).wait()

        @pl.when(t + 1 < SB // W)
        def _():
            nxt = pl.ds(abase + (t + 1) * W, W)
            pltpu.make_async_copy(row_hbm.at[nxt], row_win.at[1 - p],
                                  wsem).wait()
            pltpu.make_async_copy(ea_hbm.at[nxt], ea_win.at[1 - p],
                                  wsem).wait()
        return carry
    lax.fori_loop(0, SB // W, _a1, 0)
    plsc.subcore_barrier()

    pltpu.sync_copy(sq_s.at[pl.ds(base, RPT)], slice_v)
    for k in range(RPT // L):
        slice_v[pl.ds(k * L, L)] = _newton_rsqrt(slice_v[pl.ds(k * L, L)])
    pltpu.sync_copy(slice_v, sq_s.at[pl.ds(base, RPT)])
    plsc.subcore_barrier()
    pltpu.sync_copy(sq_s, rsq_v)

    pltpu.sync_copy(row_hbm.at[pl.ds(abase, W)], row_win.at[0])
    pltpu.sync_copy(ea_hbm.at[pl.ds(abase, W)], ea_win.at[0])

    def _a2(t, carry):
        p = t % 2

        @pl.when(t + 1 < SB // W)
        def _():
            nxt = pl.ds(abase + (t + 1) * W, W)
            pltpu.async_copy(row_hbm.at[nxt], row_win.at[1 - p], wsem)
            pltpu.async_copy(ea_hbm.at[nxt], ea_win.at[1 - p], wsem)
        for b in range(W):
            for g in range(8):
                rowv = row_win[p, b, pl.ds(g * L, L)]
                eav = ea_win[p, b, pl.ds(g * L, L)]
                rsqv = plsc.load_gather(rsq_v, [rowv])
                wbuf[b, pl.ds(g * L, L)] = jnp.exp(bvec * eav * rsqv)
        for b in range(W):
            pltpu.async_copy(wbuf.at[b], asum_s.at[row_win.at[p, b]], ssem,
                             add=True)
        for b in range(W):
            pltpu.make_async_copy(wbuf.at[b], asum_s.at[row_win.at[p, b]],
                                  ssem).wait()

        @pl.when(t + 1 < SB // W)
        def _():
            nxt = pl.ds(abase + (t + 1) * W, W)
            pltpu.make_async_copy(row_hbm.at[nxt], row_win.at[1 - p],
                                  wsem).wait()
            pltpu.make_async_copy(ea_hbm.at[nxt], ea_win.at[1 - p],
                                  wsem).wait()
        return carry
    lax.fori_loop(0, SB // W, _a2, 0)
    plsc.subcore_barrier()

    pltpu.sync_copy(asum_s.at[pl.ds(base, RPT)], slice_v)
    epsv = eps_v[...]
    for k in range(RPT // L):
        slice_v[pl.ds(k * L, L)] = 1.0 / (slice_v[pl.ds(k * L, L)] + 1e-16)

    @pl.when(c == 0)
    def _():
        pltpu.sync_copy(slice_v, wi_hbm.at[pl.ds(base, RPT)])
    for k in range(RPT // L):
        slice_v[pl.ds(k * L, L)] = (1.0 + epsv) + ebeta * slice_v[pl.ds(k * L, L)]

    @pl.when(c == 0)
    def _():
        pltpu.sync_copy(slice_v, ws_hbm.at[pl.ds(base, RPT)])

    pltpu.sync_copy(row_hbm.at[pl.ds(gbase, W)], row_win.at[0])
    pltpu.sync_copy(ea_hbm.at[pl.ds(gbase, W)], ea_win.at[0])
    pltpu.sync_copy(col_hbm.at[pl.ds(gbase, W)], col_win.at[0])
    pltpu.async_copy(xn_hbm.at[col_win.at[0, 0]], gbuf.at[0], gsem)

    def _bwin(t, carry):
        p = t % 2

        @pl.when(t + 1 < NWB)
        def _():
            nxt = pl.ds(gbase + (t + 1) * W, W)
            pltpu.async_copy(row_hbm.at[nxt], row_win.at[1 - p], wsem)
            pltpu.async_copy(ea_hbm.at[nxt], ea_win.at[1 - p], wsem)
            pltpu.async_copy(col_hbm.at[nxt], col_win.at[1 - p], wsem)

        for b in range(W):
            jb = t * W + b
            q = b % 2
            if b == W - 1:
                @pl.when(t + 1 < NWB)
                def _():
                    nxt = pl.ds(gbase + (t + 1) * W, W)
                    pltpu.make_async_copy(row_hbm.at[nxt], row_win.at[1 - p],
                                          wsem).wait()
                    pltpu.make_async_copy(ea_hbm.at[nxt], ea_win.at[1 - p],
                                          wsem).wait()
                    pltpu.make_async_copy(col_hbm.at[nxt], col_win.at[1 - p],
                                          wsem).wait()
            @pl.when(jb + 1 < BB)
            def _():
                pn = p if b + 1 < W else 1 - p
                bn = (b + 1) % W
                pltpu.async_copy(xn_hbm.at[col_win.at[pn, bn]],
                                 gbuf.at[1 - q], gsem)
            for g in range(8):
                rowv = row_win[p, b, pl.ds(g * L, L)]
                eav = ea_win[p, b, pl.ds(g * L, L)]
                rsqv = plsc.load_gather(rsq_v, [rowv])
                wbuf[0, pl.ds(g * L, L)] = jnp.exp(bvec * eav * rsqv)
            pltpu.make_async_copy(xn_hbm.at[col_win.at[p, b]],
                                  gbuf.at[q], gsem).wait()

            def _grpfn(g, carry2):
                wv = wbuf[0, pl.ds(g * L, L)]
                for u in range(L):
                    wb = jnp.broadcast_to(wv[u], (L,))
                    j = g * L + u
                    for k in range(D // L):
                        gbuf[q, j, pl.ds(k * L, L)] = \
                            gbuf[q, j, pl.ds(k * L, L)] * wb
                return carry2
            lax.fori_loop(0, 8, _grpfn, 0)
            pltpu.sync_copy(gbuf.at[q], acc_s.at[row_win.at[p, b]],
                            add=True)
        return carry
    lax.fori_loop(0, NWB, _bwin, 0)
    plsc.subcore_barrier()

    @pl.when(c == 0)
    def _():
        pltpu.sync_copy(acc_s.at[pl.ds(base, RPT)],
                        p0_hbm.at[pl.ds(base, RPT)])

    @pl.when(c == 1)
    def _():
        pltpu.sync_copy(acc_s.at[pl.ds(base, RPT)],
                        p1_hbm.at[pl.ds(base, RPT)])


_sc_kernel = functools.partial(
    pl.kernel,
    out_type=(
        jax.ShapeDtypeStruct((NP, D), jnp.float32),
        jax.ShapeDtypeStruct((NP, D), jnp.float32),
        jax.ShapeDtypeStruct((NP,), jnp.float32),
        jax.ShapeDtypeStruct((NP,), jnp.float32),
    ),
    mesh=plsc.VectorSubcoreMesh(core_axis_name="c", subcore_axis_name="s"),
    compiler_params=pltpu.CompilerParams(needs_layout_passes=False),
    scratch_types=[
        pltpu.VMEM((2, W, 128), jnp.int32),
        pltpu.VMEM((2, W, 128), jnp.float32),
        pltpu.VMEM((2, W, 128), jnp.int32),
        pltpu.VMEM((NP,), jnp.float32),
        pltpu.VMEM((RPT,), jnp.float32),
        pltpu.VMEM((W, 128), jnp.float32),
        pltpu.VMEM((L,), jnp.float32),
        pltpu.VMEM((L,), jnp.float32),
        pltpu.VMEM((2, 128, D), jnp.float32),
        pltpu.VMEM_SHARED((NP,), jnp.float32),
        pltpu.VMEM_SHARED((NP,), jnp.float32),
        pltpu.VMEM_SHARED((NP, D), jnp.float32),
        pltpu.SemaphoreType.DMA,
        pltpu.SemaphoreType.DMA,
        pltpu.SemaphoreType.DMA,
    ],
)(_sc_body)


@jax.jit
def kernel(x, edge_index, edge_attr, beta, eps):
    row = edge_index[0]
    col = edge_index[1]
    pad = EP - E
    pad_row = N + (jnp.arange(pad, dtype=jnp.int32) % (NP - N))
    pad_col = jnp.arange(pad, dtype=jnp.int32) % N
    row_p = jnp.concatenate([row, pad_row]).reshape(NB, 128)
    col_p = jnp.concatenate([col, pad_col]).reshape(NB, 128)
    ea_p = jnp.concatenate(
        [edge_attr, jnp.zeros((pad,), jnp.float32)]).reshape(NB, 128)
    beta16 = jnp.broadcast_to(beta.astype(jnp.float32), (L,))
    eps16 = jnp.broadcast_to(eps.astype(jnp.float32), (L,))

    xn = pl.pallas_call(
        _xn_body,
        grid=(N // 1000,),
        in_specs=[pl.BlockSpec((1000, D), lambda i: (i, 0))],
        out_specs=pl.BlockSpec((1000, D), lambda i: (i, 0)),
        out_shape=jax.ShapeDtypeStruct((N, D), jnp.float32),
    )(x)

    p0, p1, wscale, winv = _sc_kernel(row_p, col_p, ea_p, beta16, eps16, xn)

    out = pl.pallas_call(
        _combine_body,
        grid=(N // 1000,),
        in_specs=[
            pl.BlockSpec((1000, 1), lambda i: (i, 0)),
            pl.BlockSpec((1000, 1), lambda i: (i, 0)),
            pl.BlockSpec((1000, D), lambda i: (i, 0)),
            pl.BlockSpec((1000, D), lambda i: (i, 0)),
            pl.BlockSpec((1000, D), lambda i: (i, 0)),
        ],
        out_specs=pl.BlockSpec((1000, D), lambda i: (i, 0)),
        out_shape=jax.ShapeDtypeStruct((N, D), jnp.float32),
    )(wscale.reshape(NP, 1), winv.reshape(NP, 1), xn, p0, p1)

    return out


# speedup vs baseline: 2.6062x; 1.1390x over previous
import functools

import jax
import jax.numpy as jnp
from jax import lax
from jax.experimental import pallas as pl
from jax.experimental.pallas import tpu as pltpu
from jax.experimental.pallas import tpu_sc as plsc

N = 10000
D = 128
E = 320000

NC = 2
NS = 16
L = 16
NP = 10240
SB = 160
BB = SB // NC
W = 4
NWB = BB // W
EP = NS * SB * 128
RPT = NP // NS
NB = EP // 128


def _newton_rsqrt(s):
    s = jnp.maximum(s, 1e-24)
    i = lax.bitcast_convert_type(s, jnp.int32)
    i = jnp.int32(0x5F3759DF) - lax.shift_right_logical(i, 1)
    y = lax.bitcast_convert_type(i, jnp.float32)
    for _ in range(4):
        y = y * (1.5 - 0.5 * s * y * y)
    return y


def _xn_body(x_ref, o_ref):
    xb = x_ref[...]
    s = jnp.sum(xb * xb, axis=1, keepdims=True)
    o_ref[...] = xb * lax.rsqrt(jnp.maximum(s, 1e-24))


def _combine_body(coef_ref, pa0_ref, pa1_ref, xn_ref, p0_ref, p1_ref,
                  o_ref):
    c0 = coef_ref[0]
    eb = coef_ref[1]
    winv = 1.0 / (pa0_ref[...] + pa1_ref[...] + 1e-16)
    o_ref[...] = ((c0 + eb * winv) * xn_ref[...]
                  + winv * (p0_ref[...] + p1_ref[...]))


def _sc_body(row_hbm, col_hbm, ea_hbm, beta_hbm, xn_hbm,
             p0_hbm, p1_hbm, pa0_hbm, pa1_hbm,
             row_win, ea_win, col_win, rsq_v, slice_v, wbuf,
             beta_v, gbuf,
             sq_s, asum_s, acc_s, gsem, wsem, ssem):
    c = lax.axis_index("c")
    s = lax.axis_index("s")
    base = s * RPT
    abase = s * SB
    gbase = s * SB + c * BB

    pltpu.sync_copy(beta_hbm, beta_v)
    bvec = beta_v[...]
    ebeta = jnp.exp(bvec)

    for k in range(RPT // L):
        slice_v[pl.ds(k * L, L)] = jnp.zeros((L,), jnp.float32)
    pltpu.sync_copy(slice_v, sq_s.at[pl.ds(base, RPT)])

    def _zrow(j, carry):
        for k in range(D // L):
            gbuf[0, j, pl.ds(k * L, L)] = jnp.zeros((L,), jnp.float32)
        return carry
    lax.fori_loop(0, 128, _zrow, 0)
    for t in range(RPT // 128):
        pltpu.sync_copy(gbuf.at[0], acc_s.at[pl.ds(base + t * 128, 128)])

    cz = jnp.broadcast_to((c == 0).astype(jnp.float32), (L,))
    for k in range(RPT // L):
        slice_v[pl.ds(k * L, L)] = ebeta * cz
    pltpu.sync_copy(slice_v, asum_s.at[pl.ds(base, RPT)])
    plsc.subcore_barrier()

    pltpu.sync_copy(row_hbm.at[pl.ds(abase, W)], row_win.at[0])
    pltpu.sync_copy(ea_hbm.at[pl.ds(abase, W)], ea_win.at[0])

    def _a1(t, carry):
        p = t % 2

        @pl.when(t + 1 < SB // W)
        def _():
            nxt = pl.ds(abase + (t + 1) * W, W)
            pltpu.async_copy(row_hbm.at[nxt], row_win.at[1 - p], wsem)
            pltpu.async_copy(ea_hbm.at[nxt], ea_win.at[1 - p], wsem)
        for b in range(W):
            for g in range(8):
                eav = ea_win[p, b, pl.ds(g * L, L)]
                wbuf[b, pl.ds(g * L, L)] = eav * eav
        for b in range(W):
            pltpu.async_copy(wbuf.at[b], sq_s.at[row_win.at[p, b]], ssem,
                             add=True)
        for b in range(W):
            pltpu.make_async_copy(wbuf.at[b], sq_s.at[row_win.at[p, b]],
                                  ssem).wait()

        @pl.when(t + 1 < SB // W)
        def _():
            nxt = pl.ds(abase + (t + 1) * W, W)
            pltpu.make_async_copy(row_hbm.at[nxt], row_win.at[1 - p],
                                  wsem).wait()
            pltpu.make_async_copy(ea_hbm.at[nxt], ea_win.at[1 - p],
                                  wsem).wait()
        return carry
    lax.fori_loop(0, SB // W, _a1, 0)
    plsc.subcore_barrier()

    pltpu.sync_copy(sq_s.at[pl.ds(base, RPT)], slice_v)
    for k in range(RPT // L):
        slice_v[pl.ds(k * L, L)] = _newton_rsqrt(slice_v[pl.ds(k * L, L)])
    pltpu.sync_copy(slice_v, sq_s.at[pl.ds(base, RPT)])
    plsc.subcore_barrier()
    pltpu.sync_copy(sq_s, rsq_v)

    pltpu.sync_copy(row_hbm.at[pl.ds(gbase, W)], row_win.at[0])
    pltpu.sync_copy(ea_hbm.at[pl.ds(gbase, W)], ea_win.at[0])
    pltpu.sync_copy(col_hbm.at[pl.ds(gbase, W)], col_win.at[0])
    pltpu.async_copy(xn_hbm.at[col_win.at[0, 0]], gbuf.at[0], gsem)

    def _bwin(t, carry):
        p = t % 2

        @pl.when(t + 1 < NWB)
        def _():
            nxt = pl.ds(gbase + (t + 1) * W, W)
            pltpu.async_copy(row_hbm.at[nxt], row_win.at[1 - p], wsem)
            pltpu.async_copy(ea_hbm.at[nxt], ea_win.at[1 - p], wsem)
            pltpu.async_copy(col_hbm.at[nxt], col_win.at[1 - p], wsem)

        for b in range(W):
            jb = t * W + b
            q = b % 2
            if b == W - 1:
                @pl.when(t + 1 < NWB)
                def _():
                    nxt = pl.ds(gbase + (t + 1) * W, W)
                    pltpu.make_async_copy(row_hbm.at[nxt], row_win.at[1 - p],
                                          wsem).wait()
                    pltpu.make_async_copy(ea_hbm.at[nxt], ea_win.at[1 - p],
                                          wsem).wait()
                    pltpu.make_async_copy(col_hbm.at[nxt], col_win.at[1 - p],
                                          wsem).wait()
            @pl.when(jb + 1 < BB)
            def _():
                pn = p if b + 1 < W else 1 - p
                bn = (b + 1) % W
                pltpu.async_copy(xn_hbm.at[col_win.at[pn, bn]],
                                 gbuf.at[1 - q], gsem)
            for g in range(8):
                rowv = row_win[p, b, pl.ds(g * L, L)]
                eav = ea_win[p, b, pl.ds(g * L, L)]
                rsqv = plsc.load_gather(rsq_v, [rowv])
                wbuf[b, pl.ds(g * L, L)] = jnp.exp(bvec * eav * rsqv)
            pltpu.async_copy(wbuf.at[b], asum_s.at[row_win.at[p, b]], ssem,
                             add=True)
            pltpu.make_async_copy(xn_hbm.at[col_win.at[p, b]],
                                  gbuf.at[q], gsem).wait()

            def _grpfn(g, carry2):
                wv = wbuf[b, pl.ds(g * L, L)]
                for u in range(L):
                    wb = jnp.broadcast_to(wv[u], (L,))
                    j = g * L + u
                    for k in range(D // L):
                        gbuf[q, j, pl.ds(k * L, L)] = \
                            gbuf[q, j, pl.ds(k * L, L)] * wb
                return carry2
            lax.fori_loop(0, 8, _grpfn, 0)
            pltpu.sync_copy(gbuf.at[q], acc_s.at[row_win.at[p, b]],
                            add=True)
        for b in range(W):
            pltpu.make_async_copy(wbuf.at[b], asum_s.at[row_win.at[p, b]],
                                  ssem).wait()
        return carry
    lax.fori_loop(0, NWB, _bwin, 0)
    plsc.subcore_barrier()

    @pl.when(c == 0)
    def _():
        pltpu.sync_copy(acc_s.at[pl.ds(base, RPT)],
                        p0_hbm.at[pl.ds(base, RPT)])
        pltpu.sync_copy(asum_s.at[pl.ds(base, RPT)],
                        pa0_hbm.at[pl.ds(base, RPT)])

    @pl.when(c == 1)
    def _():
        pltpu.sync_copy(acc_s.at[pl.ds(base, RPT)],
                        p1_hbm.at[pl.ds(base, RPT)])
        pltpu.sync_copy(asum_s.at[pl.ds(base, RPT)],
                        pa1_hbm.at[pl.ds(base, RPT)])


_sc_kernel = functools.partial(
    pl.kernel,
    out_type=(
        jax.ShapeDtypeStruct((NP, D), jnp.float32),
        jax.ShapeDtypeStruct((NP, D), jnp.float32),
        jax.ShapeDtypeStruct((NP,), jnp.float32),
        jax.ShapeDtypeStruct((NP,), jnp.float32),
    ),
    mesh=plsc.VectorSubcoreMesh(core_axis_name="c", subcore_axis_name="s"),
    compiler_params=pltpu.CompilerParams(needs_layout_passes=False),
    scratch_types=[
        pltpu.VMEM((2, W, 128), jnp.int32),
        pltpu.VMEM((2, W, 128), jnp.float32),
        pltpu.VMEM((2, W, 128), jnp.int32),
        pltpu.VMEM((NP,), jnp.float32),
        pltpu.VMEM((RPT,), jnp.float32),
        pltpu.VMEM((W, 128), jnp.float32),
        pltpu.VMEM((L,), jnp.float32),
        pltpu.VMEM((2, 128, D), jnp.float32),
        pltpu.VMEM_SHARED((NP,), jnp.float32),
        pltpu.VMEM_SHARED((NP,), jnp.float32),
        pltpu.VMEM_SHARED((NP, D), jnp.float32),
        pltpu.SemaphoreType.DMA,
        pltpu.SemaphoreType.DMA,
        pltpu.SemaphoreType.DMA,
    ],
)(_sc_body)


@jax.jit
def kernel(x, edge_index, edge_attr, beta, eps):
    row = edge_index[0]
    col = edge_index[1]
    pad = EP - E
    pad_row = N + (jnp.arange(pad, dtype=jnp.int32) % (NP - N))
    pad_col = jnp.arange(pad, dtype=jnp.int32) % N
    row_p = jnp.concatenate([row, pad_row]).reshape(NB, 128)
    col_p = jnp.concatenate([col, pad_col]).reshape(NB, 128)
    ea_p = jnp.concatenate(
        [edge_attr, jnp.zeros((pad,), jnp.float32)]).reshape(NB, 128)
    beta16 = jnp.broadcast_to(beta.astype(jnp.float32), (L,))
    coef = jnp.stack([1.0 + eps.astype(jnp.float32)[0],
                      jnp.exp(beta.astype(jnp.float32)[0])])

    xn = pl.pallas_call(
        _xn_body,
        grid=(N // 1000,),
        in_specs=[pl.BlockSpec((1000, D), lambda i: (i, 0))],
        out_specs=pl.BlockSpec((1000, D), lambda i: (i, 0)),
        out_shape=jax.ShapeDtypeStruct((N, D), jnp.float32),
    )(x)

    p0, p1, pa0, pa1 = _sc_kernel(row_p, col_p, ea_p, beta16, xn)

    out = pl.pallas_call(
        _combine_body,
        grid=(N // 1000,),
        in_specs=[
            pl.BlockSpec(memory_space=pltpu.SMEM),
            pl.BlockSpec((1000, 1), lambda i: (i, 0)),
            pl.BlockSpec((1000, 1), lambda i: (i, 0)),
            pl.BlockSpec((1000, D), lambda i: (i, 0)),
            pl.BlockSpec((1000, D), lambda i: (i, 0)),
            pl.BlockSpec((1000, D), lambda i: (i, 0)),
        ],
        out_specs=pl.BlockSpec((1000, D), lambda i: (i, 0)),
        out_shape=jax.ShapeDtypeStruct((N, D), jnp.float32),
    )(coef, pa0.reshape(NP, 1), pa1.reshape(NP, 1), xn, p0, p1)

    return out


# speedup vs baseline: 2.6104x; 1.0016x over previous
import functools

import jax
import jax.numpy as jnp
from jax import lax
from jax.experimental import pallas as pl
from jax.experimental.pallas import tpu as pltpu
from jax.experimental.pallas import tpu_sc as plsc

N = 10000
D = 128
E = 320000

NC = 2
NS = 16
L = 16
NP = 10240
SB = 160
BB = SB // NC
W = 4
NWB = BB // W
EP = NS * SB * 128
RPT = NP // NS
NB = EP // 128


def _newton_rsqrt(s):
    s = jnp.maximum(s, 1e-24)
    i = lax.bitcast_convert_type(s, jnp.int32)
    i = jnp.int32(0x5F3759DF) - lax.shift_right_logical(i, 1)
    y = lax.bitcast_convert_type(i, jnp.float32)
    for _ in range(4):
        y = y * (1.5 - 0.5 * s * y * y)
    return y


def _xn_body(x_ref, o_ref):
    xb = x_ref[...]
    s = jnp.sum(xb * xb, axis=1, keepdims=True)
    o_ref[...] = xb * lax.rsqrt(jnp.maximum(s, 1e-24))


def _combine_body(coef_ref, pa0_ref, pa1_ref, xn_ref, p0_ref, p1_ref,
                  o_ref):
    c0 = coef_ref[0]
    eb = coef_ref[1]
    winv = 1.0 / (pa0_ref[...] + pa1_ref[...] + 1e-16)
    o_ref[...] = ((c0 + eb * winv) * xn_ref[...]
                  + winv * (p0_ref[...] + p1_ref[...]))


def _sc_body(row_hbm, col_hbm, ea_hbm, beta_hbm, xn_hbm,
             p0_hbm, p1_hbm, pa0_hbm, pa1_hbm,
             row_win, ea_win, col_win, rsq_v, slice_v, wbuf,
             beta_v, gbuf,
             sq_s, asum_s, acc_s, gsem, wsem, ssem, asem):
    c = lax.axis_index("c")
    s = lax.axis_index("s")
    base = s * RPT
    abase = s * SB
    gbase = s * SB + c * BB

    pltpu.sync_copy(beta_hbm, beta_v)
    bvec = beta_v[...]
    ebeta = jnp.exp(bvec)

    for k in range(RPT // L):
        slice_v[pl.ds(k * L, L)] = jnp.zeros((L,), jnp.float32)
    pltpu.sync_copy(slice_v, sq_s.at[pl.ds(base, RPT)])

    def _zrow(j, carry):
        for k in range(D // L):
            gbuf[0, j, pl.ds(k * L, L)] = jnp.zeros((L,), jnp.float32)
        return carry
    lax.fori_loop(0, 128, _zrow, 0)
    for t in range(RPT // 128):
        pltpu.sync_copy(gbuf.at[0], acc_s.at[pl.ds(base + t * 128, 128)])

    cz = jnp.broadcast_to((c == 0).astype(jnp.float32), (L,))
    for k in range(RPT // L):
        slice_v[pl.ds(k * L, L)] = ebeta * cz
    pltpu.sync_copy(slice_v, asum_s.at[pl.ds(base, RPT)])
    plsc.subcore_barrier()

    pltpu.sync_copy(row_hbm.at[pl.ds(abase, W)], row_win.at[0])
    pltpu.sync_copy(ea_hbm.at[pl.ds(abase, W)], ea_win.at[0])

    def _a1(t, carry):
        p = t % 2

        @pl.when(t + 1 < SB // W)
        def _():
            nxt = pl.ds(abase + (t + 1) * W, W)
            pltpu.async_copy(row_hbm.at[nxt], row_win.at[1 - p], wsem)
            pltpu.async_copy(ea_hbm.at[nxt], ea_win.at[1 - p], wsem)
        for b in range(W):
            for g in range(8):
                eav = ea_win[p, b, pl.ds(g * L, L)]
                wbuf[b, pl.ds(g * L, L)] = eav * eav
        for b in range(W):
            pltpu.async_copy(wbuf.at[b], sq_s.at[row_win.at[p, b]], ssem,
                             add=True)
        for b in range(W):
            pltpu.make_async_copy(wbuf.at[b], sq_s.at[row_win.at[p, b]],
                                  ssem).wait()

        @pl.when(t + 1 < SB // W)
        def _():
            nxt = pl.ds(abase + (t + 1) * W, W)
            pltpu.make_async_copy(row_hbm.at[nxt], row_win.at[1 - p],
                                  wsem).wait()
            pltpu.make_async_copy(ea_hbm.at[nxt], ea_win.at[1 - p],
                                  wsem).wait()
        return carry
    lax.fori_loop(0, SB // W, _a1, 0)
    plsc.subcore_barrier()

    pltpu.sync_copy(sq_s.at[pl.ds(base, RPT)], slice_v)
    for k in range(RPT // L):
        slice_v[pl.ds(k * L, L)] = _newton_rsqrt(slice_v[pl.ds(k * L, L)])
    pltpu.sync_copy(slice_v, sq_s.at[pl.ds(base, RPT)])
    plsc.subcore_barrier()
    pltpu.sync_copy(sq_s, rsq_v)

    pltpu.sync_copy(row_hbm.at[pl.ds(gbase, W)], row_win.at[0])
    pltpu.sync_copy(ea_hbm.at[pl.ds(gbase, W)], ea_win.at[0])
    pltpu.sync_copy(col_hbm.at[pl.ds(gbase, W)], col_win.at[0])
    pltpu.async_copy(xn_hbm.at[col_win.at[0, 0]], gbuf.at[0], gsem)

    def _bwin(t, carry):
        p = t % 2

        for b in range(W):
            jb = t * W + b
            q = b % 2
            for g in range(8):
                rowv = row_win[p, b, pl.ds(g * L, L)]
                eav = ea_win[p, b, pl.ds(g * L, L)]
                rsqv = plsc.load_gather(rsq_v, [rowv])
                wbuf[b, pl.ds(g * L, L)] = jnp.exp(bvec * eav * rsqv)
            pltpu.async_copy(wbuf.at[b], asum_s.at[row_win.at[p, b]], asem,
                             add=True)
            if b == 0:
                @pl.when(t > 0)
                def _():
                    pltpu.make_async_copy(gbuf.at[1 - q],
                                          acc_s.at[row_win.at[p, b]],
                                          ssem).wait()
                @pl.when(t + 1 < NWB)
                def _():
                    nxt = pl.ds(gbase + (t + 1) * W, W)
                    pltpu.async_copy(row_hbm.at[nxt], row_win.at[1 - p],
                                     wsem)
                    pltpu.async_copy(ea_hbm.at[nxt], ea_win.at[1 - p], wsem)
                    pltpu.async_copy(col_hbm.at[nxt], col_win.at[1 - p],
                                     wsem)
            else:
                pltpu.make_async_copy(gbuf.at[1 - q],
                                      acc_s.at[row_win.at[p, b]],
                                      ssem).wait()
            if b == W - 1:
                @pl.when(t + 1 < NWB)
                def _():
                    nxt = pl.ds(gbase + (t + 1) * W, W)
                    pltpu.make_async_copy(row_hbm.at[nxt], row_win.at[1 - p],
                                          wsem).wait()
                    pltpu.make_async_copy(ea_hbm.at[nxt], ea_win.at[1 - p],
                                          wsem).wait()
                    pltpu.make_async_copy(col_hbm.at[nxt], col_win.at[1 - p],
                                          wsem).wait()
            @pl.when(jb + 1 < BB)
            def _():
                pn = p if b + 1 < W else 1 - p
                bn = (b + 1) % W
                pltpu.async_copy(xn_hbm.at[col_win.at[pn, bn]],
                                 gbuf.at[1 - q], gsem)
            pltpu.make_async_copy(xn_hbm.at[col_win.at[p, b]],
                                  gbuf.at[q], gsem).wait()

            def _grpfn(g, carry2):
                wv = wbuf[b, pl.ds(g * L, L)]
                for u in range(L):
                    wb = jnp.broadcast_to(wv[u], (L,))
                    j = g * L + u
                    for k in range(D // L):
                        gbuf[q, j, pl.ds(k * L, L)] = \
                            gbuf[q, j, pl.ds(k * L, L)] * wb
                return carry2
            lax.fori_loop(0, 8, _grpfn, 0)
            pltpu.async_copy(gbuf.at[q], acc_s.at[row_win.at[p, b]], ssem,
                             add=True)
        for b in range(W):
            pltpu.make_async_copy(wbuf.at[b], asum_s.at[row_win.at[p, b]],
                                  asem).wait()
        return carry
    lax.fori_loop(0, NWB, _bwin, 0)
    pltpu.make_async_copy(gbuf.at[0], acc_s.at[row_win.at[0, 0]],
                          ssem).wait()
    plsc.subcore_barrier()

    @pl.when(c == 0)
    def _():
        pltpu.sync_copy(acc_s.at[pl.ds(base, RPT)],
                        p0_hbm.at[pl.ds(base, RPT)])
        pltpu.sync_copy(asum_s.at[pl.ds(base, RPT)],
                        pa0_hbm.at[pl.ds(base, RPT)])

    @pl.when(c == 1)
    def _():
        pltpu.sync_copy(acc_s.at[pl.ds(base, RPT)],
                        p1_hbm.at[pl.ds(base, RPT)])
        pltpu.sync_copy(asum_s.at[pl.ds(base, RPT)],
                        pa1_hbm.at[pl.ds(base, RPT)])


_sc_kernel = functools.partial(
    pl.kernel,
    out_type=(
        jax.ShapeDtypeStruct((NP, D), jnp.float32),
        jax.ShapeDtypeStruct((NP, D), jnp.float32),
        jax.ShapeDtypeStruct((NP,), jnp.float32),
        jax.ShapeDtypeStruct((NP,), jnp.float32),
    ),
    mesh=plsc.VectorSubcoreMesh(core_axis_name="c", subcore_axis_name="s"),
    compiler_params=pltpu.CompilerParams(needs_layout_passes=False),
    scratch_types=[
        pltpu.VMEM((2, W, 128), jnp.int32),
        pltpu.VMEM((2, W, 128), jnp.float32),
        pltpu.VMEM((2, W, 128), jnp.int32),
        pltpu.VMEM((NP,), jnp.float32),
        pltpu.VMEM((RPT,), jnp.float32),
        pltpu.VMEM((W, 128), jnp.float32),
        pltpu.VMEM((L,), jnp.float32),
        pltpu.VMEM((2, 128, D), jnp.float32),
        pltpu.VMEM_SHARED((NP,), jnp.float32),
        pltpu.VMEM_SHARED((NP,), jnp.float32),
        pltpu.VMEM_SHARED((NP, D), jnp.float32),
        pltpu.SemaphoreType.DMA,
        pltpu.SemaphoreType.DMA,
        pltpu.SemaphoreType.DMA,
        pltpu.SemaphoreType.DMA,
    ],
)(_sc_body)


@jax.jit
def kernel(x, edge_index, edge_attr, beta, eps):
    row = edge_index[0]
    col = edge_index[1]
    pad = EP - E
    pad_row = N + (jnp.arange(pad, dtype=jnp.int32) % (NP - N))
    pad_col = jnp.arange(pad, dtype=jnp.int32) % N
    row_p = jnp.concatenate([row, pad_row]).reshape(NB, 128)
    col_p = jnp.concatenate([col, pad_col]).reshape(NB, 128)
    ea_p = jnp.concatenate(
        [edge_attr, jnp.zeros((pad,), jnp.float32)]).reshape(NB, 128)
    beta16 = jnp.broadcast_to(beta.astype(jnp.float32), (L,))
    coef = jnp.stack([1.0 + eps.astype(jnp.float32)[0],
                      jnp.exp(beta.astype(jnp.float32)[0])])

    xn = pl.pallas_call(
        _xn_body,
        grid=(N // 1000,),
        in_specs=[pl.BlockSpec((1000, D), lambda i: (i, 0))],
        out_specs=pl.BlockSpec((1000, D), lambda i: (i, 0)),
        out_shape=jax.ShapeDtypeStruct((N, D), jnp.float32),
    )(x)

    p0, p1, pa0, pa1 = _sc_kernel(row_p, col_p, ea_p, beta16, xn)

    out = pl.pallas_call(
        _combine_body,
        grid=(N // 1000,),
        in_specs=[
            pl.BlockSpec(memory_space=pltpu.SMEM),
            pl.BlockSpec((1000, 1), lambda i: (i, 0)),
            pl.BlockSpec((1000, 1), lambda i: (i, 0)),
            pl.BlockSpec((1000, D), lambda i: (i, 0)),
            pl.BlockSpec((1000, D), lambda i: (i, 0)),
            pl.BlockSpec((1000, D), lambda i: (i, 0)),
        ],
        out_specs=pl.BlockSpec((1000, D), lambda i: (i, 0)),
        out_shape=jax.ShapeDtypeStruct((N, D), jnp.float32),
    )(coef, pa0.reshape(NP, 1), pa1.reshape(NP, 1), xn, p0, p1)

    return out
